# baseline (device time: 698256 ns/iter reference)
import jax
import jax.numpy as jnp
from jax import lax
from jax.experimental import pallas as pl
from jax.experimental.pallas import tpu as pltpu

N_DEV = 8
N_PASS = 2


def _gelu(y):
    c = 0.7978845608028654
    return 0.5 * y * (1.0 + jnp.tanh(c * (y + 0.044715 * y * y * y)))


def kernel(x, w_mat):
    m_tot, _ = x.shape
    _, n = w_mat.shape
    m_per = m_tot // N_DEV
    n_q = n // 4

    def body(x_ref, w_ref, out_ref, sendR, sendL, recvR, recvL,
             send_semR, send_semL, recv_semsR, recv_semsL,
             out_semR, out_semL, creditR, creditL):
        my = lax.axis_index("i")
        left = lax.rem(my + N_DEV - 1, N_DEV)
        right = lax.rem(my + 1, N_DEV)

        barrier = pltpu.get_barrier_semaphore()
        for nbr in (left, right):
            pl.semaphore_signal(barrier, inc=1, device_id=(nbr,),
                                device_id_type=pl.DeviceIdType.MESH)
        pl.semaphore_wait(barrier, 2)

        def partial(c, q):
            xc = x_ref[pl.ds(c * m_per, m_per), :]
            wq = w_ref[:, q * n_q:(q + 1) * n_q]
            return jnp.dot(xc, wq, preferred_element_type=jnp.float32)

        def rdma(src, dst_slots, slot, send_sem, recv_sems, to):
            return pltpu.make_async_remote_copy(
                src_ref=src,
                dst_ref=dst_slots.at[slot],
                send_sem=send_sem,
                recv_sem=recv_sems.at[slot],
                device_id=(to,),
                device_id_type=pl.DeviceIdType.MESH,
            )

        rings = (
            (sendR, recvR, send_semR, recv_semsR, right, left, creditR, out_semR),
            (sendL, recvL, send_semL, recv_semsL, left, right, creditL, out_semL),
        )
        pending = [None, None]

        for jq in range(N_PASS):
            for t in range(N_DEV - 1):
                k = jq * (N_DEV - 1) + t
                cR = lax.rem(my + N_DEV - 1 - t, N_DEV)
                cL = lax.rem(my + 1 + t, N_DEV)
                parts = (partial(cR, jq), partial(cL, 2 + jq))
                for r, (sbuf, rslots, ssem, rsems, to, upstream, credit,
                        _osem) in enumerate(rings):
                    p = parts[r]
                    if t == 0:
                        sbuf[...] = p
                    else:
                        pending[r].wait()
                        sbuf[...] = rslots[(k - 1) % 2] + p
                        if k <= 12:
                            pl.semaphore_signal(
                                credit, inc=1, device_id=(upstream,),
                                device_id_type=pl.DeviceIdType.MESH)
                    if k >= 2:
                        pl.semaphore_wait(credit, 1)
                    pending[r] = rdma(sbuf, rslots, k % 2, ssem, rsems, to)
                    pending[r].start()

            k_last = jq * (N_DEV - 1) + N_DEV - 2
            parts = (partial(my, jq), partial(my, 2 + jq))
            copies = []
            for r, (sbuf, rslots, _ssem, _rsems, _to, upstream, credit,
                    osem) in enumerate(rings):
                q = (jq, 2 + jq)[r]
                pending[r].wait()
                pending[r] = None
                sbuf[...] = _gelu(rslots[k_last % 2] + parts[r])
                if jq < N_PASS - 1:
                    pl.semaphore_signal(
                        credit, inc=1, device_id=(upstream,),
                        device_id_type=pl.DeviceIdType.MESH)
                cp = pltpu.make_async_copy(
                    sbuf, out_ref.at[:, q * n_q:(q + 1) * n_q], osem)
                cp.start()
                copies.append(cp)
            for cp in copies:
                cp.wait()

    return pl.pallas_call(
        body,
        out_shape=jax.ShapeDtypeStruct((m_per, n), jnp.float32),
        in_specs=[
            pl.BlockSpec(memory_space=pltpu.VMEM),
            pl.BlockSpec(memory_space=pltpu.VMEM),
        ],
        out_specs=pl.BlockSpec(memory_space=pl.ANY),
        scratch_shapes=[
            pltpu.VMEM((m_per, n_q), jnp.float32),
            pltpu.VMEM((m_per, n_q), jnp.float32),
            pltpu.VMEM((2, m_per, n_q), jnp.float32),
            pltpu.VMEM((2, m_per, n_q), jnp.float32),
            pltpu.SemaphoreType.DMA,
            pltpu.SemaphoreType.DMA,
            pltpu.SemaphoreType.DMA((2,)),
            pltpu.SemaphoreType.DMA((2,)),
            pltpu.SemaphoreType.DMA,
            pltpu.SemaphoreType.DMA,
            pltpu.SemaphoreType.REGULAR,
            pltpu.SemaphoreType.REGULAR,
        ],
        compiler_params=pltpu.CompilerParams(
            collective_id=0,
            vmem_limit_bytes=60 * 1024 * 1024,
        ),
    )(x, w_mat)
